# Initial kernel scaffold; baseline (speedup 1.0000x reference)
import jax
import jax.numpy as jnp
from jax.experimental import pallas as pl


def kernel(x, x_sub, pos, pos_sub, W_lin, W_src, W_dst, W_pos, b_pos, W_attn, b_attn, W_mlp, b_mlp, gamma, beta):
    raise NotImplementedError("write your pallas kernel here")


# speedup vs baseline: 6.4172x; 6.4172x over previous
import functools

import jax
import jax.numpy as jnp
from jax import lax
from jax.experimental import pallas as pl
from jax.experimental.pallas import tpu as pltpu
from jax.experimental.pallas import tpu_sc as plsc

N = 50000
NS = 12500
C = 128
K = 8
SP = 12544
R_KNN = 200
R_ATT = 200
R_MOM = 2000
B_PAD = 401408
GW = B_PAD // 32
GCHUNK = 256
TD = 256

def _knn_body(pos_ref, ps_ref, out_ref):
    t = pos_ref[...]
    s = ps_ref[...]
    dx = t[:, 0:1] - s[0:1, :]
    dy = t[:, 1:2] - s[1:2, :]
    dz = t[:, 2:3] - s[2:3, :]
    d = (dx * dx + dy * dy) + dz * dz
    col = lax.broadcasted_iota(jnp.int32, d.shape, 1)
    cols = []
    for _ in range(K):
        m = jnp.min(d, axis=1, keepdims=True)
        j = jnp.min(jnp.where(d == m, col, 0x7FFFFFFF), axis=1, keepdims=True)
        d = jnp.where(col == j, float("inf"), d)
        cols.append(j)
    out_ref[...] = jnp.concatenate(cols, axis=1)


def _knn_call(pos8, psT):
    return pl.pallas_call(
        _knn_body,
        grid=(N // R_KNN,),
        in_specs=[
            pl.BlockSpec((R_KNN, 8), lambda i: (i, 0)),
            pl.BlockSpec((8, SP), lambda i: (0, 0)),
        ],
        out_specs=pl.BlockSpec((R_KNN, K), lambda i: (i, 0)),
        out_shape=jax.ShapeDtypeStruct((N, K), jnp.int32),
    )(pos8, psT)


def _tbl_body(xsub_ref, ps_ref, wpos_ref, out_ref):
    out_ref[:, 0:C] = xsub_ref[...]
    out_ref[:, C:TD] = jnp.dot(ps_ref[...], wpos_ref[...],
                               preferred_element_type=jnp.float32)


def _tbl_call(xsub_pad, ps_pad8, wpos8):
    return pl.pallas_call(
        _tbl_body,
        grid=(8,),
        in_specs=[
            pl.BlockSpec((SP // 8, C), lambda i: (i, 0)),
            pl.BlockSpec((SP // 8, 8), lambda i: (i, 0)),
            pl.BlockSpec((8, C), lambda i: (0, 0)),
        ],
        out_specs=pl.BlockSpec((SP // 8, TD), lambda i: (i, 0)),
        out_shape=jax.ShapeDtypeStruct((SP, TD), jnp.float32),
    )(xsub_pad, ps_pad8, wpos8)


def _mom_body(x_ref, w_ref, b_ref, out_ref):
    step = pl.program_id(0)
    h = jnp.dot(x_ref[...], w_ref[...], preferred_element_type=jnp.float32)
    h = h + b_ref[0:1, :]
    s = jnp.sum(h, axis=0, keepdims=True)
    q = jnp.sum(h * h, axis=0, keepdims=True)

    @pl.when(step == 0)
    def _():
        out_ref[...] = jnp.zeros_like(out_ref)

    out_ref[0:1, :] += s
    out_ref[1:2, :] += q

    @pl.when(step == pl.num_programs(0) - 1)
    def _():
        mean = out_ref[0:1, :] / N
        var = out_ref[1:2, :] / N - mean * mean
        out_ref[0:1, :] = mean
        out_ref[1:2, :] = lax.rsqrt(var + 1e-5)


def _mom_call(x, w_mlp, b_mlp8):
    return pl.pallas_call(
        _mom_body,
        grid=(N // R_MOM,),
        in_specs=[
            pl.BlockSpec((R_MOM, C), lambda i: (i, 0)),
            pl.BlockSpec((C, C), lambda i: (0, 0)),
            pl.BlockSpec((8, C), lambda i: (0, 0)),
        ],
        out_specs=pl.BlockSpec((8, C), lambda i: (0, 0)),
        out_shape=jax.ShapeDtypeStruct((8, C), jnp.float32),
    )(x, w_mlp, b_mlp8)


def _att_body(x_ref, pos_ref, g_ref, wsrc_ref, wlin_ref, wdst_ref,
              wpos_ref, wattn_ref, wmlp_ref, vecs_ref, stats_ref, out_ref):
    R = R_ATT
    RK = R * K
    xb = x_ref[...]
    g = g_ref[:, 0:C]
    bsrc = g_ref[:, C:TD]
    ad = jnp.dot(xb, wdst_ref[...], preferred_element_type=jnp.float32)
    adE = jnp.broadcast_to(ad.reshape(R, 1, C), (R, K, C)).reshape(RK, C)
    a_srcE = jnp.dot(g, wsrc_ref[...], preferred_element_type=jnp.float32)
    xsE = jnp.dot(g, wlin_ref[...], preferred_element_type=jnp.float32)
    pb = pos_ref[...]
    pdst = jnp.dot(pb, wpos_ref[...], preferred_element_type=jnp.float32)
    pdstE = jnp.broadcast_to(pdst.reshape(R, 1, C), (R, K, C)).reshape(RK, C)
    dpre = pdstE - bsrc + vecs_ref[0:1, :]
    delta = dpre * jax.nn.sigmoid(dpre)
    alpha = adE - a_srcE + delta
    ap = jnp.dot(alpha, wattn_ref[...], preferred_element_type=jnp.float32)
    ap = ap + vecs_ref[1:2, :]
    alpha = ap * jax.nn.sigmoid(ap)
    a3 = alpha.reshape(R, K, C)
    amax = jnp.max(a3, axis=1, keepdims=True)
    aexp = jnp.exp(a3 - amax)
    asum = jnp.sum(aexp, axis=1, keepdims=True)
    attn = aexp / (asum + 1e-16)
    msg = attn * (xsE + delta).reshape(R, K, C)
    x_interp = jnp.sum(msg, axis=1)
    h = jnp.dot(xb, wmlp_ref[...], preferred_element_type=jnp.float32)
    h = h + vecs_ref[2:3, :]
    hn = (h - stats_ref[0:1, :]) * stats_ref[1:2, :] * vecs_ref[3:4, :] \
        + vecs_ref[4:5, :]
    out_ref[...] = hn * jax.nn.sigmoid(hn) + x_interp


def _att_call(x, pos8, g, W_src, W_lin, W_dst, wpos8, W_attn, W_mlp,
              vecs, stats):
    R = R_ATT
    return pl.pallas_call(
        _att_body,
        grid=(N // R,),
        in_specs=[
            pl.BlockSpec((R, C), lambda i: (i, 0)),
            pl.BlockSpec((R, 8), lambda i: (i, 0)),
            pl.BlockSpec((R * K, TD), lambda i: (i, 0)),
            pl.BlockSpec((C, C), lambda i: (0, 0)),
            pl.BlockSpec((C, C), lambda i: (0, 0)),
            pl.BlockSpec((C, C), lambda i: (0, 0)),
            pl.BlockSpec((8, C), lambda i: (0, 0)),
            pl.BlockSpec((C, C), lambda i: (0, 0)),
            pl.BlockSpec((C, C), lambda i: (0, 0)),
            pl.BlockSpec((8, C), lambda i: (0, 0)),
            pl.BlockSpec((8, C), lambda i: (0, 0)),
        ],
        out_specs=pl.BlockSpec((R, C), lambda i: (i, 0)),
        out_shape=jax.ShapeDtypeStruct((N, C), jnp.float32),
    )(x, pos8, g, W_src, W_lin, W_dst, wpos8, W_attn, W_mlp, vecs, stats)


def _gather_sc(tbl, idx_flat):
    mesh = plsc.VectorSubcoreMesh(core_axis_name="c", subcore_axis_name="s")

    @functools.partial(
        pl.kernel,
        mesh=mesh,
        out_type=jax.ShapeDtypeStruct((B_PAD, TD), jnp.float32),
        scratch_types=[
            pltpu.VMEM((GW,), jnp.int32),
            pltpu.VMEM((GCHUNK, TD), jnp.float32),
            pltpu.SemaphoreType.DMA,
        ],
    )
    def k(tbl_hbm, idx_hbm, g_hbm, idx_v, r1, s1):
        wid = lax.axis_index("s") * 2 + lax.axis_index("c")
        base = wid * GW
        pltpu.sync_copy(idx_hbm.at[pl.ds(base, GW)], idx_v)

        def body(i, carry):
            sl = pl.ds(i * GCHUNK, GCHUNK)
            pltpu.async_copy(tbl_hbm.at[idx_v.at[sl]], r1, s1).wait()
            osl = pl.ds(base + i * GCHUNK, GCHUNK)
            pltpu.sync_copy(r1, g_hbm.at[osl])
            return carry

        lax.fori_loop(0, GW // GCHUNK, body, 0)

    return k(tbl, idx_flat)


def kernel(x, x_sub, pos, pos_sub, W_lin, W_src, W_dst, W_pos, b_pos,
           W_attn, b_attn, W_mlp, b_mlp, gamma, beta):
    pos8 = jnp.pad(pos, ((0, 0), (0, 5)))
    psT = jnp.pad(pos_sub.T, ((0, 5), (0, SP - NS)), constant_values=1e3)
    idx = _knn_call(pos8, psT)

    idx_flat = jnp.concatenate(
        [idx.reshape(-1), jnp.zeros((B_PAD - N * K,), jnp.int32)])
    wpos8 = jnp.pad(W_pos, ((0, 5), (0, 0)))
    xsub_pad = jnp.pad(x_sub, ((0, SP - NS), (0, 0)))
    ps_pad8 = jnp.pad(pos_sub, ((0, SP - NS), (0, 5)))
    tbl = _tbl_call(xsub_pad, ps_pad8, wpos8)
    g = _gather_sc(tbl, idx_flat)

    b_mlp8 = jnp.broadcast_to(b_mlp[None, :], (8, C))
    stats = _mom_call(x, W_mlp, b_mlp8)

    z = jnp.zeros_like(b_pos)
    vecs = jnp.stack([b_pos, b_attn, b_mlp, gamma, beta, z, z, z])

    return _att_call(x, pos8, g, W_src, W_lin, W_dst, wpos8, W_attn,
                     W_mlp, vecs, stats)


# speedup vs baseline: 8.5103x; 1.3262x over previous
import functools

import jax
import jax.numpy as jnp
from jax import lax
from jax.experimental import pallas as pl
from jax.experimental.pallas import tpu as pltpu
from jax.experimental.pallas import tpu_sc as plsc

N = 50000
NS = 12500
C = 128
K = 8
SP = 12544
R_KNN = 80
W_FOLD = 256
R_ATT = 200
R_MOM = 2000
B_PAD = 401408
GW = B_PAD // 32
GCHUNK = 256
TD = 256

def _knn_body(pos_ref, ps_ref, out_ref):
    t = pos_ref[...]
    s = ps_ref[...]
    dx = t[:, 0:1] - s[0:1, :]
    dy = t[:, 1:2] - s[1:2, :]
    dz = t[:, 2:3] - s[2:3, :]
    d = (dx * dx + dy * dy) + dz * dz
    b = lax.bitcast_convert_type(d, jnp.int32)
    nsl = SP // W_FOLD
    MAXI = 0x7FFFFFFF

    def tree_min(lst):
        while len(lst) > 1:
            nxt = [jnp.minimum(a, c) for a, c in zip(lst[::2], lst[1::2])]
            if len(lst) % 2:
                nxt.append(lst[-1])
            lst = nxt
        return lst[0]

    vs = [(b[:, si * W_FOLD:(si + 1) * W_FOLD] & ~0x3F) | si
          for si in range(nsl)]
    m1 = tree_min(vs)
    vs = [jnp.where(v == m1, MAXI, v) for v in vs]
    m2 = tree_min(vs)
    vs = [jnp.where(v == m2, MAXI, v) for v in vs]
    m3 = tree_min(vs)
    cand = jnp.concatenate([m1, m2, m3], axis=1)
    col = lax.broadcasted_iota(jnp.int32, cand.shape, 1)
    cols = []
    for _ in range(K):
        m = jnp.min(cand, axis=1, keepdims=True)
        p = jnp.min(jnp.where(cand == m, col, MAXI), axis=1, keepdims=True)
        cand = jnp.where(col == p, MAXI, cand)
        cols.append((m & 0x3F) * W_FOLD + (p & (W_FOLD - 1)))
    out_ref[...] = jnp.concatenate(cols, axis=1)


def _knn_call(pos8, psT):
    return pl.pallas_call(
        _knn_body,
        grid=(N // R_KNN,),
        in_specs=[
            pl.BlockSpec((R_KNN, 8), lambda i: (i, 0)),
            pl.BlockSpec((8, SP), lambda i: (0, 0)),
        ],
        out_specs=pl.BlockSpec((R_KNN, K), lambda i: (i, 0)),
        out_shape=jax.ShapeDtypeStruct((N, K), jnp.int32),
    )(pos8, psT)


def _tbl_body(xsub_ref, ps_ref, wpos_ref, out_ref):
    out_ref[:, 0:C] = xsub_ref[...]
    out_ref[:, C:TD] = jnp.dot(ps_ref[...], wpos_ref[...],
                               preferred_element_type=jnp.float32)


def _tbl_call(xsub_pad, ps_pad8, wpos8):
    return pl.pallas_call(
        _tbl_body,
        grid=(8,),
        in_specs=[
            pl.BlockSpec((SP // 8, C), lambda i: (i, 0)),
            pl.BlockSpec((SP // 8, 8), lambda i: (i, 0)),
            pl.BlockSpec((8, C), lambda i: (0, 0)),
        ],
        out_specs=pl.BlockSpec((SP // 8, TD), lambda i: (i, 0)),
        out_shape=jax.ShapeDtypeStruct((SP, TD), jnp.float32),
    )(xsub_pad, ps_pad8, wpos8)


def _mom_body(x_ref, w_ref, b_ref, out_ref):
    step = pl.program_id(0)
    h = jnp.dot(x_ref[...], w_ref[...], preferred_element_type=jnp.float32)
    h = h + b_ref[0:1, :]
    s = jnp.sum(h, axis=0, keepdims=True)
    q = jnp.sum(h * h, axis=0, keepdims=True)

    @pl.when(step == 0)
    def _():
        out_ref[...] = jnp.zeros_like(out_ref)

    out_ref[0:1, :] += s
    out_ref[1:2, :] += q

    @pl.when(step == pl.num_programs(0) - 1)
    def _():
        mean = out_ref[0:1, :] / N
        var = out_ref[1:2, :] / N - mean * mean
        out_ref[0:1, :] = mean
        out_ref[1:2, :] = lax.rsqrt(var + 1e-5)


def _mom_call(x, w_mlp, b_mlp8):
    return pl.pallas_call(
        _mom_body,
        grid=(N // R_MOM,),
        in_specs=[
            pl.BlockSpec((R_MOM, C), lambda i: (i, 0)),
            pl.BlockSpec((C, C), lambda i: (0, 0)),
            pl.BlockSpec((8, C), lambda i: (0, 0)),
        ],
        out_specs=pl.BlockSpec((8, C), lambda i: (0, 0)),
        out_shape=jax.ShapeDtypeStruct((8, C), jnp.float32),
    )(x, w_mlp, b_mlp8)


def _att_body(x_ref, pos_ref, g_ref, wsrc_ref, wlin_ref, wdst_ref,
              wpos_ref, wattn_ref, wmlp_ref, vecs_ref, stats_ref, out_ref):
    R = R_ATT
    RK = R * K
    xb = x_ref[...]
    g = g_ref[:, 0:C]
    bsrc = g_ref[:, C:TD]
    ad = jnp.dot(xb, wdst_ref[...], preferred_element_type=jnp.float32)
    adE = jnp.broadcast_to(ad.reshape(R, 1, C), (R, K, C)).reshape(RK, C)
    a_srcE = jnp.dot(g, wsrc_ref[...], preferred_element_type=jnp.float32)
    xsE = jnp.dot(g, wlin_ref[...], preferred_element_type=jnp.float32)
    pb = pos_ref[...]
    pdst = jnp.dot(pb, wpos_ref[...], preferred_element_type=jnp.float32)
    pdstE = jnp.broadcast_to(pdst.reshape(R, 1, C), (R, K, C)).reshape(RK, C)
    dpre = pdstE - bsrc + vecs_ref[0:1, :]
    delta = dpre * jax.nn.sigmoid(dpre)
    alpha = adE - a_srcE + delta
    ap = jnp.dot(alpha, wattn_ref[...], preferred_element_type=jnp.float32)
    ap = ap + vecs_ref[1:2, :]
    alpha = ap * jax.nn.sigmoid(ap)
    a3 = alpha.reshape(R, K, C)
    amax = jnp.max(a3, axis=1, keepdims=True)
    aexp = jnp.exp(a3 - amax)
    asum = jnp.sum(aexp, axis=1, keepdims=True)
    attn = aexp / (asum + 1e-16)
    msg = attn * (xsE + delta).reshape(R, K, C)
    x_interp = jnp.sum(msg, axis=1)
    h = jnp.dot(xb, wmlp_ref[...], preferred_element_type=jnp.float32)
    h = h + vecs_ref[2:3, :]
    hn = (h - stats_ref[0:1, :]) * stats_ref[1:2, :] * vecs_ref[3:4, :] \
        + vecs_ref[4:5, :]
    out_ref[...] = hn * jax.nn.sigmoid(hn) + x_interp


def _att_call(x, pos8, g, W_src, W_lin, W_dst, wpos8, W_attn, W_mlp,
              vecs, stats):
    R = R_ATT
    return pl.pallas_call(
        _att_body,
        grid=(N // R,),
        in_specs=[
            pl.BlockSpec((R, C), lambda i: (i, 0)),
            pl.BlockSpec((R, 8), lambda i: (i, 0)),
            pl.BlockSpec((R * K, TD), lambda i: (i, 0)),
            pl.BlockSpec((C, C), lambda i: (0, 0)),
            pl.BlockSpec((C, C), lambda i: (0, 0)),
            pl.BlockSpec((C, C), lambda i: (0, 0)),
            pl.BlockSpec((8, C), lambda i: (0, 0)),
            pl.BlockSpec((C, C), lambda i: (0, 0)),
            pl.BlockSpec((C, C), lambda i: (0, 0)),
            pl.BlockSpec((8, C), lambda i: (0, 0)),
            pl.BlockSpec((8, C), lambda i: (0, 0)),
        ],
        out_specs=pl.BlockSpec((R, C), lambda i: (i, 0)),
        out_shape=jax.ShapeDtypeStruct((N, C), jnp.float32),
    )(x, pos8, g, W_src, W_lin, W_dst, wpos8, W_attn, W_mlp, vecs, stats)


def _gather_sc(tbl, idx_flat):
    mesh = plsc.VectorSubcoreMesh(core_axis_name="c", subcore_axis_name="s")

    @functools.partial(
        pl.kernel,
        mesh=mesh,
        out_type=jax.ShapeDtypeStruct((B_PAD, TD), jnp.float32),
        scratch_types=[
            pltpu.VMEM((GW,), jnp.int32),
            pltpu.VMEM((GCHUNK, TD), jnp.float32),
            pltpu.SemaphoreType.DMA,
        ],
    )
    def k(tbl_hbm, idx_hbm, g_hbm, idx_v, r1, s1):
        wid = lax.axis_index("s") * 2 + lax.axis_index("c")
        base = wid * GW
        pltpu.sync_copy(idx_hbm.at[pl.ds(base, GW)], idx_v)

        def body(i, carry):
            sl = pl.ds(i * GCHUNK, GCHUNK)
            pltpu.async_copy(tbl_hbm.at[idx_v.at[sl]], r1, s1).wait()
            osl = pl.ds(base + i * GCHUNK, GCHUNK)
            pltpu.sync_copy(r1, g_hbm.at[osl])
            return carry

        lax.fori_loop(0, GW // GCHUNK, body, 0)

    return k(tbl, idx_flat)


def kernel(x, x_sub, pos, pos_sub, W_lin, W_src, W_dst, W_pos, b_pos,
           W_attn, b_attn, W_mlp, b_mlp, gamma, beta):
    pos8 = jnp.pad(pos, ((0, 0), (0, 5)))
    psT = jnp.pad(pos_sub.T, ((0, 5), (0, SP - NS)), constant_values=1e3)
    idx = _knn_call(pos8, psT)

    idx_flat = jnp.concatenate(
        [idx.reshape(-1), jnp.zeros((B_PAD - N * K,), jnp.int32)])
    wpos8 = jnp.pad(W_pos, ((0, 5), (0, 0)))
    xsub_pad = jnp.pad(x_sub, ((0, SP - NS), (0, 0)))
    ps_pad8 = jnp.pad(pos_sub, ((0, SP - NS), (0, 5)))
    tbl = _tbl_call(xsub_pad, ps_pad8, wpos8)
    g = _gather_sc(tbl, idx_flat)

    b_mlp8 = jnp.broadcast_to(b_mlp[None, :], (8, C))
    stats = _mom_call(x, W_mlp, b_mlp8)

    z = jnp.zeros_like(b_pos)
    vecs = jnp.stack([b_pos, b_attn, b_mlp, gamma, beta, z, z, z])

    return _att_call(x, pos8, g, W_src, W_lin, W_dst, wpos8, W_attn,
                     W_mlp, vecs, stats)


# speedup vs baseline: 10.4374x; 1.2264x over previous
import functools

import jax
import jax.numpy as jnp
from jax import lax
from jax.experimental import pallas as pl
from jax.experimental.pallas import tpu as pltpu
from jax.experimental.pallas import tpu_sc as plsc

N = 50000
NS = 12500
C = 128
K = 8
SP = 12544
R_KNN = 80
W_FOLD = 256
R_ATT = 200
R_MOM = 2000
B_PAD = 401408
GW = B_PAD // 32
GCHUNK = 128
TD = 256

def _knn_body(pos_ref, aux_ref, out_ref):
    t = pos_ref[...]
    aux = aux_ref[...]
    g = jnp.dot(t, aux[0:8, :], preferred_element_type=jnp.float32)
    tn = jnp.sum(t * t, axis=1, keepdims=True)
    d = jnp.maximum(tn + (aux[8:9, :] - g), 0.0)
    b = lax.bitcast_convert_type(d, jnp.int32)
    nsl = SP // W_FOLD
    MAXI = 0x7FFFFFFF

    def tree_min(lst):
        while len(lst) > 1:
            nxt = [jnp.minimum(a, c) for a, c in zip(lst[::2], lst[1::2])]
            if len(lst) % 2:
                nxt.append(lst[-1])
            lst = nxt
        return lst[0]

    vs = [(b[:, si * W_FOLD:(si + 1) * W_FOLD] & ~0x3F) | si
          for si in range(nsl)]
    m1 = tree_min(vs)
    vs = [jnp.where(v == m1, MAXI, v) for v in vs]
    m2 = tree_min(vs)
    vs = [jnp.where(v == m2, MAXI, v) for v in vs]
    m3 = tree_min(vs)
    cand = jnp.concatenate([m1, m2, m3], axis=1)
    col = lax.broadcasted_iota(jnp.int32, cand.shape, 1)
    cols = []
    for _ in range(K):
        m = jnp.min(cand, axis=1, keepdims=True)
        p = jnp.min(jnp.where(cand == m, col, MAXI), axis=1, keepdims=True)
        cand = jnp.where(col == p, MAXI, cand)
        cols.append((m & 0x3F) * W_FOLD + (p & (W_FOLD - 1)))
    out_ref[...] = jnp.concatenate(cols, axis=1)


def _knn_call(pos8, aux):
    return pl.pallas_call(
        _knn_body,
        grid=(N // R_KNN,),
        in_specs=[
            pl.BlockSpec((R_KNN, 8), lambda i: (i, 0)),
            pl.BlockSpec((16, SP), lambda i: (0, 0)),
        ],
        out_specs=pl.BlockSpec((R_KNN, K), lambda i: (i, 0)),
        out_shape=jax.ShapeDtypeStruct((N, K), jnp.int32),
    )(pos8, aux)


def _tbl_body(xsub_ref, ps_ref, wpos_ref, out_ref):
    out_ref[:, 0:C] = xsub_ref[...]
    out_ref[:, C:TD] = jnp.dot(ps_ref[...], wpos_ref[...],
                               preferred_element_type=jnp.float32)


def _tbl_call(xsub_pad, ps_pad8, wpos8):
    return pl.pallas_call(
        _tbl_body,
        grid=(8,),
        in_specs=[
            pl.BlockSpec((SP // 8, C), lambda i: (i, 0)),
            pl.BlockSpec((SP // 8, 8), lambda i: (i, 0)),
            pl.BlockSpec((8, C), lambda i: (0, 0)),
        ],
        out_specs=pl.BlockSpec((SP // 8, TD), lambda i: (i, 0)),
        out_shape=jax.ShapeDtypeStruct((SP, TD), jnp.float32),
    )(xsub_pad, ps_pad8, wpos8)


def _mom_body(x_ref, w_ref, b_ref, out_ref):
    step = pl.program_id(0)
    h = jnp.dot(x_ref[...], w_ref[...], preferred_element_type=jnp.float32)
    h = h + b_ref[0:1, :]
    s = jnp.sum(h, axis=0, keepdims=True)
    q = jnp.sum(h * h, axis=0, keepdims=True)

    @pl.when(step == 0)
    def _():
        out_ref[...] = jnp.zeros_like(out_ref)

    out_ref[0:1, :] += s
    out_ref[1:2, :] += q

    @pl.when(step == pl.num_programs(0) - 1)
    def _():
        mean = out_ref[0:1, :] / N
        var = out_ref[1:2, :] / N - mean * mean
        out_ref[0:1, :] = mean
        out_ref[1:2, :] = lax.rsqrt(var + 1e-5)


def _mom_call(x, w_mlp, b_mlp8):
    return pl.pallas_call(
        _mom_body,
        grid=(N // R_MOM,),
        in_specs=[
            pl.BlockSpec((R_MOM, C), lambda i: (i, 0)),
            pl.BlockSpec((C, C), lambda i: (0, 0)),
            pl.BlockSpec((8, C), lambda i: (0, 0)),
        ],
        out_specs=pl.BlockSpec((8, C), lambda i: (0, 0)),
        out_shape=jax.ShapeDtypeStruct((8, C), jnp.float32),
    )(x, w_mlp, b_mlp8)


def _att_body(x_ref, pos_ref, g_ref, wsrc_ref, wlin_ref, wdst_ref,
              wpos_ref, wattn_ref, wmlp_ref, vecs_ref, stats_ref, out_ref):
    R = R_ATT
    RK = R * K
    xb = x_ref[...]
    g = g_ref[:, 0:C]
    bsrc = g_ref[:, C:TD]
    ad = jnp.dot(xb, wdst_ref[...], preferred_element_type=jnp.float32)
    adE = jnp.broadcast_to(ad.reshape(R, 1, C), (R, K, C)).reshape(RK, C)
    a_srcE = jnp.dot(g, wsrc_ref[...], preferred_element_type=jnp.float32)
    xsE = jnp.dot(g, wlin_ref[...], preferred_element_type=jnp.float32)
    pb = pos_ref[...]
    pdst = jnp.dot(pb, wpos_ref[...], preferred_element_type=jnp.float32)
    pdstE = jnp.broadcast_to(pdst.reshape(R, 1, C), (R, K, C)).reshape(RK, C)
    dpre = pdstE - bsrc + vecs_ref[0:1, :]
    delta = dpre * jax.nn.sigmoid(dpre)
    alpha = adE - a_srcE + delta
    ap = jnp.dot(alpha, wattn_ref[...], preferred_element_type=jnp.float32)
    ap = ap + vecs_ref[1:2, :]
    alpha = ap * jax.nn.sigmoid(ap)
    a3 = alpha.reshape(R, K, C)
    amax = jnp.max(a3, axis=1, keepdims=True)
    aexp = jnp.exp(a3 - amax)
    asum = jnp.sum(aexp, axis=1, keepdims=True)
    attn = aexp / (asum + 1e-16)
    msg = attn * (xsE + delta).reshape(R, K, C)
    x_interp = jnp.sum(msg, axis=1)
    h = jnp.dot(xb, wmlp_ref[...], preferred_element_type=jnp.float32)
    h = h + vecs_ref[2:3, :]
    hn = (h - stats_ref[0:1, :]) * stats_ref[1:2, :] * vecs_ref[3:4, :] \
        + vecs_ref[4:5, :]
    out_ref[...] = hn * jax.nn.sigmoid(hn) + x_interp


def _att_call(x, pos8, g, W_src, W_lin, W_dst, wpos8, W_attn, W_mlp,
              vecs, stats):
    R = R_ATT
    return pl.pallas_call(
        _att_body,
        grid=(N // R,),
        in_specs=[
            pl.BlockSpec((R, C), lambda i: (i, 0)),
            pl.BlockSpec((R, 8), lambda i: (i, 0)),
            pl.BlockSpec((R * K, TD), lambda i: (i, 0)),
            pl.BlockSpec((C, C), lambda i: (0, 0)),
            pl.BlockSpec((C, C), lambda i: (0, 0)),
            pl.BlockSpec((C, C), lambda i: (0, 0)),
            pl.BlockSpec((8, C), lambda i: (0, 0)),
            pl.BlockSpec((C, C), lambda i: (0, 0)),
            pl.BlockSpec((C, C), lambda i: (0, 0)),
            pl.BlockSpec((8, C), lambda i: (0, 0)),
            pl.BlockSpec((8, C), lambda i: (0, 0)),
        ],
        out_specs=pl.BlockSpec((R, C), lambda i: (i, 0)),
        out_shape=jax.ShapeDtypeStruct((N, C), jnp.float32),
    )(x, pos8, g, W_src, W_lin, W_dst, wpos8, W_attn, W_mlp, vecs, stats)


def _gather_sc(tbl, idx_flat):
    mesh = plsc.VectorSubcoreMesh(core_axis_name="c", subcore_axis_name="s")

    @functools.partial(
        pl.kernel,
        mesh=mesh,
        out_type=jax.ShapeDtypeStruct((B_PAD, TD), jnp.float32),
        scratch_types=[
            pltpu.VMEM((GW,), jnp.int32),
            pltpu.VMEM((GCHUNK, TD), jnp.float32),
            pltpu.VMEM((GCHUNK, TD), jnp.float32),
            pltpu.SemaphoreType.DMA,
            pltpu.SemaphoreType.DMA,
        ],
    )
    def k(tbl_hbm, idx_hbm, g_hbm, idx_v, ra, rb, sa, sb):
        wid = lax.axis_index("s") * 2 + lax.axis_index("c")
        base = wid * GW
        nch = GW // GCHUNK
        pltpu.sync_copy(idx_hbm.at[pl.ds(base, GW)], idx_v)

        def gat(ch, buf, sem):
            return pltpu.async_copy(
                tbl_hbm.at[idx_v.at[pl.ds(ch * GCHUNK, GCHUNK)]], buf, sem)

        def put(ch, buf):
            pltpu.sync_copy(buf, g_hbm.at[pl.ds(base + ch * GCHUNK, GCHUNK)])

        gat(0, ra, sa)

        def body(i, carry):
            ca = 2 * i
            gat(ca + 1, rb, sb)
            sa_copy = pltpu.make_async_copy(
                tbl_hbm.at[idx_v.at[pl.ds(ca * GCHUNK, GCHUNK)]], ra, sa)
            sa_copy.wait()
            put(ca, ra)

            @pl.when(ca + 2 < nch)
            def _():
                gat(ca + 2, ra, sa)

            pltpu.make_async_copy(
                tbl_hbm.at[idx_v.at[pl.ds((ca + 1) * GCHUNK, GCHUNK)]],
                rb, sb).wait()
            put(ca + 1, rb)
            return carry

        lax.fori_loop(0, nch // 2, body, 0)

    return k(tbl, idx_flat)


def kernel(x, x_sub, pos, pos_sub, W_lin, W_src, W_dst, W_pos, b_pos,
           W_attn, b_attn, W_mlp, b_mlp, gamma, beta):
    pc = pos - 0.5
    sc = pos_sub - 0.5
    pos8 = jnp.pad(pos, ((0, 0), (0, 5)))
    pos8c = jnp.pad(pc, ((0, 0), (0, 5)))
    psT2 = jnp.pad(2.0 * sc.T, ((0, 5), (0, SP - NS)),
                   constant_values=2e3)
    snorm = jnp.pad(jnp.sum(sc * sc, axis=1)[None, :],
                    ((0, 7), (0, SP - NS)), constant_values=3e6)
    aux = jnp.concatenate([psT2, snorm], axis=0)
    idx = _knn_call(pos8c, aux)

    idx_flat = jnp.concatenate(
        [idx.reshape(-1), jnp.zeros((B_PAD - N * K,), jnp.int32)])
    wpos8 = jnp.pad(W_pos, ((0, 5), (0, 0)))
    xsub_pad = jnp.pad(x_sub, ((0, SP - NS), (0, 0)))
    ps_pad8 = jnp.pad(pos_sub, ((0, SP - NS), (0, 5)))
    tbl = _tbl_call(xsub_pad, ps_pad8, wpos8)
    g = _gather_sc(tbl, idx_flat)

    b_mlp8 = jnp.broadcast_to(b_mlp[None, :], (8, C))
    stats = _mom_call(x, W_mlp, b_mlp8)

    z = jnp.zeros_like(b_pos)
    vecs = jnp.stack([b_pos, b_attn, b_mlp, gamma, beta, z, z, z])

    return _att_call(x, pos8, g, W_src, W_lin, W_dst, wpos8, W_attn,
                     W_mlp, vecs, stats)
